# baseline (speedup 1.0000x reference)
import functools

import jax
import jax.numpy as jnp
from jax.experimental import pallas as pl
from jax.experimental.pallas import tpu as pltpu


def _sage_kernel(
    adj_ref,
    x_ref,
    wt1_ref,
    wb1_ref,
    b1_ref,
    wc2_ref,
    bc2_ref,
    o_ref,
    y1_ref,
    hw2_ref,
    y2_ref,
    outp_ref,
    deg_ref,
    u_ref,
    dacc_ref,
    abuf_ref,
    *,
    bm,
    bw,
    nj,
):
    t = pl.program_id(0)
    i = pl.program_id(1)
    j = pl.program_id(2)
    c = o_ref.shape[1]
    m = (i * bm) // bw
    rows = pl.ds(i * bm, bm)
    cols = pl.ds(j * bw, bw)

    @pl.when((t == 0) & (i == 0) & (j == 0))
    def _():
        y1_ref[...] = jnp.dot(
            x_ref[...], wb1_ref[...], preferred_element_type=jnp.float32
        ).astype(jnp.bfloat16)

    @pl.when(t == 0)
    def _():
        @pl.when(j == 0)
        def _():
            u_ref[...] = jnp.zeros_like(u_ref)
            dacc_ref[...] = jnp.zeros_like(dacc_ref)
            outp_ref[rows, :] = jnp.zeros_like(outp_ref[rows, :])

        a = adj_ref[...].reshape(bm, bw)
        abuf_ref[...] = a.astype(jnp.bfloat16)
        dacc_ref[...] += jnp.sum(a, axis=1, keepdims=True)
        u_ref[...] += jnp.dot(
            abuf_ref[...], y1_ref[cols, :], preferred_element_type=jnp.float32
        )

        @pl.when(j < m)
        def _():
            outp_ref[rows, :] += jnp.dot(
                abuf_ref[...], y2_ref[cols, :],
                preferred_element_type=jnp.float32,
            )

        @pl.when(j == nj - 1)
        def _():
            deg = dacc_ref[...] + 1e-8
            h = (
                jnp.dot(
                    x_ref[rows, :], wt1_ref[...],
                    preferred_element_type=jnp.float32,
                )
                + b1_ref[...]
                + u_ref[...] / deg
            )
            p2 = (
                jnp.dot(h, wc2_ref[...], preferred_element_type=jnp.float32)
                + bc2_ref[...]
            )
            hw2_ref[rows, :] = p2[:, :c]
            y2_ref[rows, :] = p2[:, c:].astype(jnp.bfloat16)
            deg_ref[rows, :] = deg

    @pl.when(t == 1)
    def _():
        @pl.when(j >= m)
        def _():
            abuf_ref[...] = adj_ref[...].reshape(bm, bw).astype(jnp.bfloat16)
            outp_ref[rows, :] += jnp.dot(
                abuf_ref[...], y2_ref[cols, :],
                preferred_element_type=jnp.float32,
            )

        @pl.when(j == nj - 1)
        def _():
            o_ref[...] = jax.nn.sigmoid(
                hw2_ref[rows, :] + outp_ref[rows, :] / deg_ref[rows, :]
            )


def kernel(x, adj, W1, b1, W2, b2):
    n, f = x.shape
    h1 = W1.shape[1]
    c = W2.shape[1]
    bm = 400 if n % 400 == 0 else n
    bw = 2000 if n % 2000 == 0 else n
    ni = n // bm
    nj = n // bw

    wt1 = W1[:f]
    wb1 = W1[f:]
    wc2 = jnp.concatenate([W2[:h1], W2[h1:]], axis=1)
    bc2 = jnp.concatenate([b2, jnp.zeros_like(b2)]).reshape(1, 2 * c)

    adj4 = adj.reshape(n, nj, 1, bw)

    def adj_index(t, i, j):
        jc = jnp.where(t == 0, j, jnp.maximum(j, (i * bm) // bw))
        return (i, jc, 0, 0)

    body = functools.partial(_sage_kernel, bm=bm, bw=bw, nj=nj)
    return pl.pallas_call(
        body,
        grid=(2, ni, nj),
        in_specs=[
            pl.BlockSpec((bm, 1, 1, bw), adj_index),
            pl.BlockSpec((n, f), lambda t, i, j: (0, 0)),
            pl.BlockSpec((f, h1), lambda t, i, j: (0, 0)),
            pl.BlockSpec((f, h1), lambda t, i, j: (0, 0)),
            pl.BlockSpec((1, h1), lambda t, i, j: (0, 0)),
            pl.BlockSpec((h1, 2 * c), lambda t, i, j: (0, 0)),
            pl.BlockSpec((1, 2 * c), lambda t, i, j: (0, 0)),
        ],
        out_specs=pl.BlockSpec((bm, c), lambda t, i, j: (i, 0)),
        out_shape=jax.ShapeDtypeStruct((n, c), jnp.float32),
        scratch_shapes=[
            pltpu.VMEM((n, h1), jnp.bfloat16),
            pltpu.VMEM((n, c), jnp.float32),
            pltpu.VMEM((n, c), jnp.bfloat16),
            pltpu.VMEM((n, c), jnp.float32),
            pltpu.VMEM((n, 1), jnp.float32),
            pltpu.VMEM((bm, h1), jnp.float32),
            pltpu.VMEM((bm, 1), jnp.float32),
            pltpu.VMEM((bm, bw), jnp.bfloat16),
        ],
        compiler_params=pltpu.CompilerParams(
            dimension_semantics=("arbitrary", "arbitrary", "arbitrary"),
        ),
    )(adj4, x, wt1, wb1, b1.reshape(1, h1), wc2, bc2)


# speedup vs baseline: 6.4465x; 6.4465x over previous
import functools

import jax
import jax.numpy as jnp
from jax.experimental import pallas as pl
from jax.experimental.pallas import tpu as pltpu


def _sage_kernel(
    adj_ref,
    adj_hbm_ref,
    x_ref,
    wt1_ref,
    wb1_ref,
    b1_ref,
    wc2_ref,
    bc2_ref,
    o_ref,
    y1_ref,
    hw2_ref,
    y2_ref,
    outp_ref,
    deg_ref,
    tail_ref,
    buf0_ref,
    buf1_ref,
    sem0,
    sem1,
    *,
    bm0,
    bm1,
    ni1,
    cw,
    nc,
    tw,
):
    t = pl.program_id(0)
    i = pl.program_id(1)
    n = x_ref.shape[0]
    c = o_ref.shape[1]

    @pl.when((t == 0) & (i == 0))
    def _():
        y1_ref[...] = jnp.dot(
            x_ref[...], wb1_ref[...], preferred_element_type=jnp.float32
        ).astype(jnp.bfloat16)

    @pl.when(t == 0)
    def _():
        rows = pl.ds(i * bm0, bm0)
        m0 = ((i // (bm1 // bm0)) * bm1) // cw

        a = adj_ref[...]
        ab = a.astype(jnp.bfloat16)
        deg = jnp.sum(a, axis=1, keepdims=True) + 1e-8
        u = jnp.dot(ab, y1_ref[...], preferred_element_type=jnp.float32)
        h = (
            jnp.dot(x_ref[rows, :], wt1_ref[...], preferred_element_type=jnp.float32)
            + b1_ref[...]
            + u / deg
        )
        p2 = jnp.dot(h, wc2_ref[...], preferred_element_type=jnp.float32) + bc2_ref[...]
        hw2_ref[rows, :] = p2[:, :c]
        y2_ref[rows, :] = p2[:, c:].astype(jnp.bfloat16)
        deg_ref[rows, :] = deg
        tail_ref[rows, :] = ab[:, nc * cw :]

        @pl.when(m0 > 0)
        def _():
            ridx = jax.lax.broadcasted_iota(jnp.int32, (n, c), 0)
            y2m = jnp.where(ridx < m0 * cw, y2_ref[...], jnp.bfloat16(0.0))
            outp_ref[rows, :] = jnp.dot(ab, y2m, preferred_element_type=jnp.float32)

        @pl.when(m0 == 0)
        def _():
            outp_ref[rows, :] = jnp.zeros((bm0, c), jnp.float32)

    @pl.when((t == 1) & (i < ni1))
    def _():
        rows = pl.ds(i * bm1, bm1)
        m1 = (i * bm1) // cw
        bufs = (buf0_ref, buf1_ref)
        sems = (sem0, sem1)

        def copy(k, buf, sem):
            pltpu.make_async_copy(
                adj_hbm_ref.at[pl.ds(i * bm1, bm1), pl.ds(k * cw, cw)],
                buf,
                sem,
            ).start()

        for k in range(nc):
            @pl.when(k == m1)
            def _(k=k):
                copy(k, bufs[k % 2], sems[k % 2])

            if k + 1 < nc:
                @pl.when(k >= m1)
                def _(k=k):
                    copy(k + 1, bufs[(k + 1) % 2], sems[(k + 1) % 2])

            @pl.when(k >= m1)
            def _(k=k):
                pltpu.make_async_copy(
                    adj_hbm_ref.at[pl.ds(i * bm1, bm1), pl.ds(k * cw, cw)],
                    bufs[k % 2],
                    sems[k % 2],
                ).wait()
                outp_ref[rows, :] += jnp.dot(
                    bufs[k % 2][...].astype(jnp.bfloat16),
                    y2_ref[pl.ds(k * cw, cw), :],
                    preferred_element_type=jnp.float32,
                )

        s = outp_ref[rows, :] + jnp.dot(
            tail_ref[rows, :],
            y2_ref[pl.ds(nc * cw, tw), :],
            preferred_element_type=jnp.float32,
        )
        o_ref[...] = jax.nn.sigmoid(hw2_ref[rows, :] + s / deg_ref[rows, :])


def kernel(x, adj, W1, b1, W2, b2):
    n, f = x.shape
    h1 = W1.shape[1]
    c = W2.shape[1]
    bm0 = 40 if n % 1000 == 0 else n
    bm1 = 1000 if n % 1000 == 0 else n
    ni0 = n // bm0
    ni1 = n // bm1
    cw = 1664
    nc = n // cw
    tw = n - nc * cw

    wt1 = W1[:f]
    wb1 = W1[f:]
    wc2 = jnp.concatenate([W2[:h1], W2[h1:]], axis=1)
    bc2 = jnp.concatenate([b2, jnp.zeros_like(b2)]).reshape(1, 2 * c)

    body = functools.partial(
        _sage_kernel, bm0=bm0, bm1=bm1, ni1=ni1, cw=cw, nc=nc, tw=tw
    )
    return pl.pallas_call(
        body,
        grid=(2, ni0),
        in_specs=[
            pl.BlockSpec((bm0, n), lambda t, i: (jnp.where(t == 0, i, ni0 - 1), 0)),
            pl.BlockSpec(memory_space=pl.ANY),
            pl.BlockSpec((n, f), lambda t, i: (0, 0)),
            pl.BlockSpec((f, h1), lambda t, i: (0, 0)),
            pl.BlockSpec((f, h1), lambda t, i: (0, 0)),
            pl.BlockSpec((1, h1), lambda t, i: (0, 0)),
            pl.BlockSpec((h1, 2 * c), lambda t, i: (0, 0)),
            pl.BlockSpec((1, 2 * c), lambda t, i: (0, 0)),
        ],
        out_specs=pl.BlockSpec(
            (bm1, c),
            lambda t, i: (jnp.where(t == 1, jnp.minimum(i, ni1 - 1), 0), 0),
        ),
        out_shape=jax.ShapeDtypeStruct((n, c), jnp.float32),
        scratch_shapes=[
            pltpu.VMEM((n, h1), jnp.bfloat16),
            pltpu.VMEM((n, c), jnp.float32),
            pltpu.VMEM((n, c), jnp.bfloat16),
            pltpu.VMEM((n, c), jnp.float32),
            pltpu.VMEM((n, 1), jnp.float32),
            pltpu.VMEM((n, tw), jnp.bfloat16),
            pltpu.VMEM((bm1, cw), jnp.float32),
            pltpu.VMEM((bm1, cw), jnp.float32),
            pltpu.SemaphoreType.DMA,
            pltpu.SemaphoreType.DMA,
        ],
        compiler_params=pltpu.CompilerParams(
            dimension_semantics=("arbitrary", "arbitrary"),
        ),
    )(adj, adj, x, wt1, wb1, b1.reshape(1, h1), wc2, bc2)
